# baseline (speedup 1.0000x reference)
import functools

import jax
import jax.numpy as jnp
from jax import lax
from jax.experimental import pallas as pl
from jax.experimental.pallas import tpu as pltpu
from jax.experimental.pallas import tpu_sc as plsc

N = 10000
E = 320000
D = 128
H = 128
MH = 256

NC = 2
NS = 16
L = 16
NW = NC * NS
NPAD = 10240
RPT = NPAD // NS
EPW = E // NW
K = 80
NCHUNK = EPW // K
ZR = 80
BR = 2048

_mesh = plsc.VectorSubcoreMesh(core_axis_name="c", subcore_axis_name="s",
                               num_cores=NC, num_subcores=NS)


def _seg_body(with_cnt, x_hbm, src_hbm, dst_hbm, pagg_hbm, pcnt_hbm,
              sagg, scnt, dst_all, rows_v0, rows_v1, rows_v2,
              sv0, sv1, sv2, z1d_v, ones_v,
              si0, sg0, ss0, sc0, si1, sg1, ss1, sc1, si2, sg2, ss2, sc2):
    cid = lax.axis_index("c")
    sid = lax.axis_index("s")
    wid = sid * NC + cid
    base_r = sid * RPT

    pltpu.sync_copy(dst_hbm.at[wid], dst_all)

    def zrow_body(r, c):
        for cb in range(D // L):
            rows_v0[r, pl.ds(cb * L, L)] = jnp.zeros((L,), jnp.float32)
        return c
    lax.fori_loop(0, K, zrow_body, 0)
    if with_cnt:
        def z1_body(r, c):
            z1d_v[pl.ds(r * L, L)] = jnp.zeros((L,), jnp.float32)
            return c
        lax.fori_loop(0, RPT // L, z1_body, 0)
        for i in range(K // L):
            ones_v[pl.ds(i * L, L)] = jnp.ones((L,), jnp.float32)

    for i in range(RPT // K):
        pltpu.sync_copy(rows_v0, sagg.at[pl.ds(base_r + i * K, K)])
    if with_cnt:
        pltpu.sync_copy(z1d_v, scnt.at[pl.ds(base_r, RPT)])
    plsc.subcore_barrier()

    bufs = ((sv0, rows_v0, si0, sg0, ss0, sc0),
            (sv1, rows_v1, si1, sg1, ss1, sc1),
            (sv2, rows_v2, si2, sg2, ss2, sc2))

    def isrc(j, buf):
        sv, rv, si, sg, ss, sc = buf
        pltpu.async_copy(src_hbm.at[wid, j], sv, si)

    def gather(j, buf):
        sv, rv, si, sg, ss, sc = buf
        pltpu.make_async_copy(src_hbm.at[wid, j], sv, si).wait()
        pltpu.async_copy(x_hbm.at[sv], rv, sg)

    def process(j, buf):
        sv, rv, si, sg, ss, sc = buf
        pltpu.make_async_copy(x_hbm.at[sv], rv, sg).wait()
        pltpu.async_copy(rv, sagg.at[dst_all.at[j]], ss, add=True)
        if with_cnt:
            pltpu.async_copy(ones_v, scnt.at[dst_all.at[j]], sc, add=True)

    def drain(j, buf):
        sv, rv, si, sg, ss, sc = buf
        pltpu.make_async_copy(rv, sagg.at[dst_all.at[j]], ss).wait()
        if with_cnt:
            pltpu.make_async_copy(ones_v, scnt.at[dst_all.at[j]], sc).wait()

    def step(jj, d, do_drain, do_isrc, do_g):
        process(jj, bufs[d])
        if do_drain:
            drain(jj - 1, bufs[(d + 2) % 3])
        if do_isrc:
            isrc(jj + 3, bufs[d])
        if do_g:
            gather(jj + 1, bufs[(d + 1) % 3])

    isrc(0, bufs[0])
    isrc(1, bufs[1])
    isrc(2, bufs[2])
    gather(0, bufs[0])
    step(0, 0, False, True, True)
    step(1, 1, True, True, True)
    step(2, 2, True, True, True)

    def triple(it, c):
        jj0 = 3 * it + 3
        step(jj0 + 0, 0, True, True, True)
        step(jj0 + 1, 1, True, True, True)
        step(jj0 + 2, 2, True, True, True)
        return c
    lax.fori_loop(0, 39, triple, 0)
    step(120, 0, True, True, True)
    step(121, 1, True, True, True)
    step(122, 2, True, False, True)
    step(123, 0, True, False, True)
    step(124, 1, True, False, False)
    drain(124, bufs[1])
    plsc.subcore_barrier()

    out_r = cid * NPAD + base_r
    pltpu.sync_copy(sagg.at[pl.ds(base_r, RPT)], pagg_hbm.at[pl.ds(out_r, RPT)])
    if with_cnt:
        pltpu.sync_copy(scnt.at[pl.ds(base_r, RPT)], pcnt_hbm.at[pl.ds(out_r, RPT)])


def _make_seg(with_cnt):
    outs = (jax.ShapeDtypeStruct((2 * NPAD, D), jnp.float32),
            jax.ShapeDtypeStruct((2 * NPAD,), jnp.float32))
    scratch = [
        pltpu.VMEM_SHARED((NPAD, D), jnp.float32),
        pltpu.VMEM_SHARED((NPAD,), jnp.float32),
        pltpu.VMEM((NCHUNK, K), jnp.int32),
        pltpu.VMEM((K, D), jnp.float32),
        pltpu.VMEM((K, D), jnp.float32),
        pltpu.VMEM((K, D), jnp.float32),
        pltpu.VMEM((K,), jnp.int32),
        pltpu.VMEM((K,), jnp.int32),
        pltpu.VMEM((K,), jnp.int32),
        pltpu.VMEM((RPT,), jnp.float32),
        pltpu.VMEM((K,), jnp.float32),
    ] + [pltpu.SemaphoreType.DMA] * 12
    return pl.kernel(functools.partial(_seg_body, with_cnt),
                     out_type=outs, mesh=_mesh, scratch_types=scratch,
                     compiler_params=pltpu.CompilerParams(
                         use_tc_tiling_on_sc=False))


_seg_with_cnt = _make_seg(True)
_seg_no_cnt = _make_seg(False)


def _edge_body(a_hbm, b_hbm, src_hbm, dst_hbm, w2_hbm, bm2_hbm, out_hbm,
               a_v0, b_v0, a_v1, b_v1, src_all, dst_all,
               out_v0, out_v1, w_v, bm2_v, t_v, sa0, sb0, sa1, sb1, so0, so1):
    cid = lax.axis_index("c")
    sid = lax.axis_index("s")
    wid = sid * NC + cid
    ebase = wid * EPW

    pltpu.sync_copy(src_hbm.at[wid], src_all)
    pltpu.sync_copy(dst_hbm.at[wid], dst_all)
    pltpu.sync_copy(w2_hbm, w_v)
    pltpu.sync_copy(bm2_hbm, bm2_v)
    bias16 = bm2_v[...]
    iota17 = lax.iota(jnp.int32, L) * 17

    bufs = ((a_v0, b_v0, out_v0, sa0, sb0, so0),
            (a_v1, b_v1, out_v1, sa1, sb1, so1))

    def start(j, buf):
        av, bv, ov, sa, sb, so = buf
        pltpu.async_copy(a_hbm.at[src_all.at[j]], av, sa)
        pltpu.async_copy(b_hbm.at[dst_all.at[j]], bv, sb)

    def finish(j, buf):
        av, bv, ov, sa, sb, so = buf
        off = ebase + j * K
        pltpu.make_async_copy(a_hbm.at[src_all.at[j]], av, sa).wait()
        pltpu.make_async_copy(b_hbm.at[dst_all.at[j]], bv, sb).wait()
        pltpu.make_async_copy(ov, out_hbm.at[pl.ds(off, K)], so).wait()

        def group(g, c2):
            goff = g * L
            def jb_body(jb, accs):
                col = jb * (2 * L)
                w32 = w_v[pl.ds(col, 2 * L)]
                nxt = []
                for e in range(L):
                    a32 = av[goff + e, pl.ds(col, 2 * L)]
                    b32 = bv[goff + e, pl.ds(col, 2 * L)]
                    p = jnp.maximum(a32 + b32, jnp.bfloat16(0)) * w32
                    p0, p1 = plsc.unpack(p, format=plsc.PackFormat.INTERLEAVED)
                    nxt.append(accs[e] + p0 + p1)
                return tuple(nxt)
            accs = lax.fori_loop(
                0, MH // (2 * L), jb_body,
                tuple(jnp.zeros((L,), jnp.float32) for _ in range(L)))
            for e in range(L):
                t_v[pl.ds(e * 17, L)] = accs[e]
            tot = jnp.zeros((L,), jnp.float32)
            for jj in range(L):
                tot = tot + plsc.load_gather(t_v, [iota17 + jj])
            ov[pl.ds(goff, L)] = tot + bias16
            return c2
        lax.fori_loop(0, K // L, group, 0)
        pltpu.async_copy(ov, out_hbm.at[pl.ds(off, K)], so)

    pltpu.async_copy(out_v0, out_hbm.at[pl.ds(ebase, K)], so0)
    pltpu.async_copy(out_v1, out_hbm.at[pl.ds(ebase + K, K)], so1)

    start(0, bufs[0])
    def pair(it, c):
        j0 = 2 * it
        start(j0 + 1, bufs[1])
        finish(j0, bufs[0])
        start(j0 + 2, bufs[0])
        finish(j0 + 1, bufs[1])
        return c
    lax.fori_loop(0, (NCHUNK - 1) // 2, pair, 0)
    finish(NCHUNK - 1, bufs[0])
    pltpu.make_async_copy(out_v0, out_hbm.at[pl.ds(ebase, K)], so0).wait()
    pltpu.make_async_copy(out_v1, out_hbm.at[pl.ds(ebase, K)], so1).wait()


_edge_head = pl.kernel(
    _edge_body,
    out_type=jax.ShapeDtypeStruct((E,), jnp.float32),
    mesh=_mesh,
    scratch_types=[
        pltpu.VMEM((K, MH), jnp.bfloat16),
        pltpu.VMEM((K, MH), jnp.bfloat16),
        pltpu.VMEM((K, MH), jnp.bfloat16),
        pltpu.VMEM((K, MH), jnp.bfloat16),
        pltpu.VMEM((NCHUNK, K), jnp.int32),
        pltpu.VMEM((NCHUNK, K), jnp.int32),
        pltpu.VMEM((K,), jnp.float32),
        pltpu.VMEM((K,), jnp.float32),
        pltpu.VMEM((MH,), jnp.bfloat16),
        pltpu.VMEM((L,), jnp.float32),
        pltpu.VMEM((L * 17,), jnp.float32),
        pltpu.SemaphoreType.DMA,
        pltpu.SemaphoreType.DMA,
        pltpu.SemaphoreType.DMA,
        pltpu.SemaphoreType.DMA,
        pltpu.SemaphoreType.DMA,
        pltpu.SemaphoreType.DMA,
    ],
    compiler_params=pltpu.CompilerParams(use_tc_tiling_on_sc=False,
                                         needs_layout_passes=False),
)


def _tc1_body(pagg_ref, pcnt_ref, x_ref, w1lT_ref, b1l_ref, w1rT_ref, h1_ref):
    cnt = pcnt_ref[0] + pcnt_ref[1]
    inv = 1.0 / jnp.maximum(cnt, 1.0)
    agg = (pagg_ref[0] + pagg_ref[1]) * inv
    h = jnp.dot(agg, w1lT_ref[...], preferred_element_type=jnp.float32)
    h = h + jnp.dot(x_ref[...], w1rT_ref[...], preferred_element_type=jnp.float32)
    h1_ref[...] = jnp.maximum(h + b1l_ref[...], 0.0)


def _tc2_body(pagg_ref, pcnt_ref, h1_ref, w2lT_ref, b2l_ref, w2rT_ref,
              wm1aT_ref, wm1bT_ref, bm1_ref, a_ref, b_ref):
    cnt = pcnt_ref[0] + pcnt_ref[1]
    inv = 1.0 / jnp.maximum(cnt, 1.0)
    agg = (pagg_ref[0] + pagg_ref[1]) * inv
    h = jnp.dot(agg, w2lT_ref[...], preferred_element_type=jnp.float32)
    h = h + jnp.dot(h1_ref[...], w2rT_ref[...], preferred_element_type=jnp.float32)
    h2 = jnp.maximum(h + b2l_ref[...], 0.0)
    a_ref[...] = jnp.dot(
        h2, wm1aT_ref[...],
        preferred_element_type=jnp.float32).astype(jnp.bfloat16)
    b_ref[...] = (jnp.dot(h2, wm1bT_ref[...], preferred_element_type=jnp.float32)
                  + bm1_ref[...]).astype(jnp.bfloat16)


def _tc1(pagg, pcnt, x, w1lT, b1l, w1rT):
    grid = NPAD // BR
    return pl.pallas_call(
        _tc1_body,
        grid=(grid,),
        in_specs=[
            pl.BlockSpec((2, BR, D), lambda i: (0, i, 0)),
            pl.BlockSpec((2, BR, 1), lambda i: (0, i, 0)),
            pl.BlockSpec((BR, D), lambda i: (i, 0)),
            pl.BlockSpec((D, H), lambda i: (0, 0)),
            pl.BlockSpec((1, H), lambda i: (0, 0)),
            pl.BlockSpec((D, H), lambda i: (0, 0)),
        ],
        out_specs=pl.BlockSpec((BR, H), lambda i: (i, 0)),
        out_shape=jax.ShapeDtypeStruct((NPAD, H), jnp.float32),
    )(pagg, pcnt, x, w1lT, b1l, w1rT)


def _tc2(pagg, pcnt, h1, w2lT, b2l, w2rT, wm1aT, wm1bT, bm1):
    grid = NPAD // BR
    return pl.pallas_call(
        _tc2_body,
        grid=(grid,),
        in_specs=[
            pl.BlockSpec((2, BR, H), lambda i: (0, i, 0)),
            pl.BlockSpec((2, BR, 1), lambda i: (0, i, 0)),
            pl.BlockSpec((BR, H), lambda i: (i, 0)),
            pl.BlockSpec((H, H), lambda i: (0, 0)),
            pl.BlockSpec((1, H), lambda i: (0, 0)),
            pl.BlockSpec((H, H), lambda i: (0, 0)),
            pl.BlockSpec((H, MH), lambda i: (0, 0)),
            pl.BlockSpec((H, MH), lambda i: (0, 0)),
            pl.BlockSpec((1, MH), lambda i: (0, 0)),
        ],
        out_specs=[
            pl.BlockSpec((BR, MH), lambda i: (i, 0)),
            pl.BlockSpec((BR, MH), lambda i: (i, 0)),
        ],
        out_shape=[
            jax.ShapeDtypeStruct((NPAD, MH), jnp.bfloat16),
            jax.ShapeDtypeStruct((NPAD, MH), jnp.bfloat16),
        ],
    )(pagg, pcnt, h1, w2lT, b2l, w2rT, wm1aT, wm1bT, bm1)


def kernel(x, edge_index, W1l, b1l, W1r, W2l, b2l, W2r, Wm1, bm1, Wm2, bm2):
    src = edge_index[0].reshape(NW, NCHUNK, K)
    dst = edge_index[1].reshape(NW, NCHUNK, K)

    pagg1, pcnt = _seg_with_cnt(x, src, dst)
    pagg1 = pagg1.reshape(2, NPAD, D)
    pcnt2 = pcnt.reshape(2, NPAD, 1)

    xp = jnp.pad(x, ((0, NPAD - N), (0, 0)))
    h1 = _tc1(pagg1, pcnt2, xp, W1l.T, b1l.reshape(1, H), W1r.T)

    pagg2, _ = _seg_no_cnt(h1, src, dst)
    pagg2 = pagg2.reshape(2, NPAD, H)

    A, B = _tc2(pagg2, pcnt2, h1, W2l.T, b2l.reshape(1, H), W2r.T,
                Wm1[:, :H].T, Wm1[:, H:].T, bm1.reshape(1, MH))

    w2 = Wm2[0].astype(jnp.bfloat16)
    bm2s = jnp.full((L,), bm2[0], jnp.float32)
    out = _edge_head(A, B, src, dst, w2, bm2s)
    return out


# speedup vs baseline: 1.0368x; 1.0368x over previous
import functools

import jax
import jax.numpy as jnp
from jax import lax
from jax.experimental import pallas as pl
from jax.experimental.pallas import tpu as pltpu
from jax.experimental.pallas import tpu_sc as plsc

N = 10000
E = 320000
D = 128
H = 128
MH = 256

NC = 2
NS = 16
L = 16
NW = NC * NS
NPAD = 10240
RPT = NPAD // NS
EPW = E // NW
K = 80
NCHUNK = EPW // K
ZR = 80
BR = 2048

_mesh = plsc.VectorSubcoreMesh(core_axis_name="c", subcore_axis_name="s",
                               num_cores=NC, num_subcores=NS)


def _seg_body(with_cnt, x_hbm, src_hbm, dst_hbm, pagg_hbm, pcnt_hbm,
              sagg, scnt, src_all, dst_all, rows_v0, rows_v1,
              z1d_v, ones_v, sg0, ss0, sc0, sg1, ss1, sc1):
    cid = lax.axis_index("c")
    sid = lax.axis_index("s")
    wid = sid * NC + cid
    base_r = sid * RPT

    pltpu.sync_copy(src_hbm.at[wid], src_all)
    pltpu.sync_copy(dst_hbm.at[wid], dst_all)

    def zrow_body(r, c):
        for cb in range(D // L):
            rows_v0[r, pl.ds(cb * L, L)] = jnp.zeros((L,), jnp.float32)
        return c
    lax.fori_loop(0, K, zrow_body, 0)
    if with_cnt:
        def z1_body(r, c):
            z1d_v[pl.ds(r * L, L)] = jnp.zeros((L,), jnp.float32)
            return c
        lax.fori_loop(0, RPT // L, z1_body, 0)
        for i in range(K // L):
            ones_v[pl.ds(i * L, L)] = jnp.ones((L,), jnp.float32)

    for i in range(RPT // K):
        pltpu.sync_copy(rows_v0, sagg.at[pl.ds(base_r + i * K, K)])
    if with_cnt:
        pltpu.sync_copy(z1d_v, scnt.at[pl.ds(base_r, RPT)])
    plsc.subcore_barrier()

    bufs = ((rows_v0, sg0, ss0, sc0), (rows_v1, sg1, ss1, sc1))

    def start_gather(j, buf):
        rv, sg, ss, sc = buf
        pltpu.async_copy(x_hbm.at[src_all.at[j]], rv, sg)

    def drain_scatter(j, buf):
        rv, sg, ss, sc = buf
        pltpu.make_async_copy(rv, sagg.at[dst_all.at[j]], ss).wait()
        if with_cnt:
            pltpu.make_async_copy(ones_v, scnt.at[dst_all.at[j]], sc).wait()

    def process(j, buf):
        rv, sg, ss, sc = buf
        pltpu.make_async_copy(x_hbm.at[src_all.at[j]], rv, sg).wait()
        pltpu.async_copy(rv, sagg.at[dst_all.at[j]], ss, add=True)
        if with_cnt:
            pltpu.async_copy(ones_v, scnt.at[dst_all.at[j]], sc, add=True)

    start_gather(0, bufs[0])
    start_gather(1, bufs[1])

    def pair(it, c):
        j0 = 2 * it
        process(j0, bufs[0])
        process(j0 + 1, bufs[1])
        drain_scatter(j0, bufs[0])
        start_gather(j0 + 2, bufs[0])
        drain_scatter(j0 + 1, bufs[1])
        start_gather(j0 + 3, bufs[1])
        return c
    lax.fori_loop(0, (NCHUNK - 3) // 2, pair, 0)
    process(NCHUNK - 3, bufs[0])
    drain_scatter(NCHUNK - 3, bufs[0])
    start_gather(NCHUNK - 1, bufs[0])
    process(NCHUNK - 2, bufs[1])
    process(NCHUNK - 1, bufs[0])
    drain_scatter(NCHUNK - 1, bufs[0])
    drain_scatter(NCHUNK - 2, bufs[1])
    plsc.subcore_barrier()

    out_r = cid * NPAD + base_r
    pltpu.sync_copy(sagg.at[pl.ds(base_r, RPT)], pagg_hbm.at[pl.ds(out_r, RPT)])
    if with_cnt:
        pltpu.sync_copy(scnt.at[pl.ds(base_r, RPT)], pcnt_hbm.at[pl.ds(out_r, RPT)])


def _make_seg(with_cnt):
    outs = (jax.ShapeDtypeStruct((2 * NPAD, D), jnp.float32),
            jax.ShapeDtypeStruct((2 * NPAD,), jnp.float32))
    scratch = [
        pltpu.VMEM_SHARED((NPAD, D), jnp.float32),
        pltpu.VMEM_SHARED((NPAD,), jnp.float32),
        pltpu.VMEM((NCHUNK, K), jnp.int32),
        pltpu.VMEM((NCHUNK, K), jnp.int32),
        pltpu.VMEM((K, D), jnp.float32),
        pltpu.VMEM((K, D), jnp.float32),
        pltpu.VMEM((RPT,), jnp.float32),
        pltpu.VMEM((K,), jnp.float32),
    ] + [pltpu.SemaphoreType.DMA] * 6
    return pl.kernel(functools.partial(_seg_body, with_cnt),
                     out_type=outs, mesh=_mesh, scratch_types=scratch,
                     compiler_params=pltpu.CompilerParams(
                         use_tc_tiling_on_sc=False))


_seg_with_cnt = _make_seg(True)
_seg_no_cnt = _make_seg(False)


def _edge_body(a_hbm, b_hbm, src_hbm, dst_hbm, w2_hbm, bm2_hbm, out_hbm,
               a_v0, b_v0, a_v1, b_v1, src_all, dst_all,
               out_v0, out_v1, w_v, bm2_v, t_v, sa0, sb0, sa1, sb1, so0, so1):
    cid = lax.axis_index("c")
    sid = lax.axis_index("s")
    wid = sid * NC + cid
    ebase = wid * EPW

    pltpu.sync_copy(src_hbm.at[wid], src_all)
    pltpu.sync_copy(dst_hbm.at[wid], dst_all)
    pltpu.sync_copy(w2_hbm, w_v)
    pltpu.sync_copy(bm2_hbm, bm2_v)
    bias16 = bm2_v[...]
    iota17 = lax.iota(jnp.int32, L) * 17

    bufs = ((a_v0, b_v0, out_v0, sa0, sb0, so0),
            (a_v1, b_v1, out_v1, sa1, sb1, so1))

    def start(j, buf):
        av, bv, ov, sa, sb, so = buf
        pltpu.async_copy(a_hbm.at[src_all.at[j]], av, sa)
        pltpu.async_copy(b_hbm.at[dst_all.at[j]], bv, sb)

    def finish(j, buf):
        av, bv, ov, sa, sb, so = buf
        off = ebase + j * K
        pltpu.make_async_copy(a_hbm.at[src_all.at[j]], av, sa).wait()
        pltpu.make_async_copy(b_hbm.at[dst_all.at[j]], bv, sb).wait()
        pltpu.make_async_copy(ov, out_hbm.at[pl.ds(off, K)], so).wait()

        def group(g, c2):
            goff = g * L
            def jb_body(jb, accs):
                col = jb * (2 * L)
                w32 = w_v[pl.ds(col, 2 * L)]
                nxt = []
                for e in range(L):
                    a32 = av[goff + e, pl.ds(col, 2 * L)]
                    b32 = bv[goff + e, pl.ds(col, 2 * L)]
                    p = jnp.maximum(a32 + b32, jnp.bfloat16(0)) * w32
                    p0, p1 = plsc.unpack(p, format=plsc.PackFormat.INTERLEAVED)
                    nxt.append(accs[e] + p0 + p1)
                return tuple(nxt)
            accs = lax.fori_loop(
                0, MH // (2 * L), jb_body,
                tuple(jnp.zeros((L,), jnp.float32) for _ in range(L)))
            for e in range(L):
                t_v[pl.ds(e * 17, L)] = accs[e]
            tot = jnp.zeros((L,), jnp.float32)
            for jj in range(L):
                tot = tot + plsc.load_gather(t_v, [iota17 + jj])
            ov[pl.ds(goff, L)] = tot + bias16
            return c2
        lax.fori_loop(0, K // L, group, 0)
        pltpu.async_copy(ov, out_hbm.at[pl.ds(off, K)], so)

    pltpu.async_copy(out_v0, out_hbm.at[pl.ds(ebase, K)], so0)
    pltpu.async_copy(out_v1, out_hbm.at[pl.ds(ebase + K, K)], so1)

    start(0, bufs[0])
    def pair(it, c):
        j0 = 2 * it
        start(j0 + 1, bufs[1])
        finish(j0, bufs[0])
        start(j0 + 2, bufs[0])
        finish(j0 + 1, bufs[1])
        return c
    lax.fori_loop(0, (NCHUNK - 1) // 2, pair, 0)
    finish(NCHUNK - 1, bufs[0])
    pltpu.make_async_copy(out_v0, out_hbm.at[pl.ds(ebase, K)], so0).wait()
    pltpu.make_async_copy(out_v1, out_hbm.at[pl.ds(ebase, K)], so1).wait()


_edge_head = pl.kernel(
    _edge_body,
    out_type=jax.ShapeDtypeStruct((E,), jnp.float32),
    mesh=_mesh,
    scratch_types=[
        pltpu.VMEM((K, MH), jnp.bfloat16),
        pltpu.VMEM((K, MH), jnp.bfloat16),
        pltpu.VMEM((K, MH), jnp.bfloat16),
        pltpu.VMEM((K, MH), jnp.bfloat16),
        pltpu.VMEM((NCHUNK, K), jnp.int32),
        pltpu.VMEM((NCHUNK, K), jnp.int32),
        pltpu.VMEM((K,), jnp.float32),
        pltpu.VMEM((K,), jnp.float32),
        pltpu.VMEM((MH,), jnp.bfloat16),
        pltpu.VMEM((L,), jnp.float32),
        pltpu.VMEM((L * 17,), jnp.float32),
        pltpu.SemaphoreType.DMA,
        pltpu.SemaphoreType.DMA,
        pltpu.SemaphoreType.DMA,
        pltpu.SemaphoreType.DMA,
        pltpu.SemaphoreType.DMA,
        pltpu.SemaphoreType.DMA,
    ],
    compiler_params=pltpu.CompilerParams(use_tc_tiling_on_sc=False,
                                         needs_layout_passes=False),
)


def _tc1_body(pagg_ref, pcnt_ref, x_ref, w1lT_ref, b1l_ref, w1rT_ref, h1_ref):
    cnt = pcnt_ref[0] + pcnt_ref[1]
    inv = 1.0 / jnp.maximum(cnt, 1.0)
    agg = (pagg_ref[0] + pagg_ref[1]) * inv
    h = jnp.dot(agg, w1lT_ref[...], preferred_element_type=jnp.float32)
    h = h + jnp.dot(x_ref[...], w1rT_ref[...], preferred_element_type=jnp.float32)
    h1_ref[...] = jnp.maximum(h + b1l_ref[...], 0.0)


def _tc2_body(pagg_ref, pcnt_ref, h1_ref, w2lT_ref, b2l_ref, w2rT_ref,
              wm1aT_ref, wm1bT_ref, bm1_ref, a_ref, b_ref):
    cnt = pcnt_ref[0] + pcnt_ref[1]
    inv = 1.0 / jnp.maximum(cnt, 1.0)
    agg = (pagg_ref[0] + pagg_ref[1]) * inv
    h = jnp.dot(agg, w2lT_ref[...], preferred_element_type=jnp.float32)
    h = h + jnp.dot(h1_ref[...], w2rT_ref[...], preferred_element_type=jnp.float32)
    h2 = jnp.maximum(h + b2l_ref[...], 0.0)
    a_ref[...] = jnp.dot(
        h2, wm1aT_ref[...],
        preferred_element_type=jnp.float32).astype(jnp.bfloat16)
    b_ref[...] = (jnp.dot(h2, wm1bT_ref[...], preferred_element_type=jnp.float32)
                  + bm1_ref[...]).astype(jnp.bfloat16)


def _tc1(pagg, pcnt, x, w1lT, b1l, w1rT):
    grid = NPAD // BR
    return pl.pallas_call(
        _tc1_body,
        grid=(grid,),
        in_specs=[
            pl.BlockSpec((2, BR, D), lambda i: (0, i, 0)),
            pl.BlockSpec((2, BR, 1), lambda i: (0, i, 0)),
            pl.BlockSpec((BR, D), lambda i: (i, 0)),
            pl.BlockSpec((D, H), lambda i: (0, 0)),
            pl.BlockSpec((1, H), lambda i: (0, 0)),
            pl.BlockSpec((D, H), lambda i: (0, 0)),
        ],
        out_specs=pl.BlockSpec((BR, H), lambda i: (i, 0)),
        out_shape=jax.ShapeDtypeStruct((NPAD, H), jnp.float32),
    )(pagg, pcnt, x, w1lT, b1l, w1rT)


def _tc2(pagg, pcnt, h1, w2lT, b2l, w2rT, wm1aT, wm1bT, bm1):
    grid = NPAD // BR
    return pl.pallas_call(
        _tc2_body,
        grid=(grid,),
        in_specs=[
            pl.BlockSpec((2, BR, H), lambda i: (0, i, 0)),
            pl.BlockSpec((2, BR, 1), lambda i: (0, i, 0)),
            pl.BlockSpec((BR, H), lambda i: (i, 0)),
            pl.BlockSpec((H, H), lambda i: (0, 0)),
            pl.BlockSpec((1, H), lambda i: (0, 0)),
            pl.BlockSpec((H, H), lambda i: (0, 0)),
            pl.BlockSpec((H, MH), lambda i: (0, 0)),
            pl.BlockSpec((H, MH), lambda i: (0, 0)),
            pl.BlockSpec((1, MH), lambda i: (0, 0)),
        ],
        out_specs=[
            pl.BlockSpec((BR, MH), lambda i: (i, 0)),
            pl.BlockSpec((BR, MH), lambda i: (i, 0)),
        ],
        out_shape=[
            jax.ShapeDtypeStruct((NPAD, MH), jnp.bfloat16),
            jax.ShapeDtypeStruct((NPAD, MH), jnp.bfloat16),
        ],
    )(pagg, pcnt, h1, w2lT, b2l, w2rT, wm1aT, wm1bT, bm1)


def kernel(x, edge_index, W1l, b1l, W1r, W2l, b2l, W2r, Wm1, bm1, Wm2, bm2):
    src = edge_index[0].reshape(NW, NCHUNK, K)
    dst = edge_index[1].reshape(NW, NCHUNK, K)

    pagg1, pcnt = _seg_with_cnt(x, src, dst)
    pagg1 = pagg1.reshape(2, NPAD, D)
    pcnt2 = pcnt.reshape(2, NPAD, 1)

    xp = jnp.pad(x, ((0, NPAD - N), (0, 0)))
    h1 = _tc1(pagg1, pcnt2, xp, W1l.T, b1l.reshape(1, H), W1r.T)

    pagg2, _ = _seg_no_cnt(h1, src, dst)
    pagg2 = pagg2.reshape(2, NPAD, H)

    A, B = _tc2(pagg2, pcnt2, h1, W2l.T, b2l.reshape(1, H), W2r.T,
                Wm1[:, :H].T, Wm1[:, H:].T, bm1.reshape(1, MH))

    w2 = Wm2[0].astype(jnp.bfloat16)
    bm2s = jnp.full((L,), bm2[0], jnp.float32)
    out = _edge_head(A, B, src, dst, w2, bm2s)
    return out


# speedup vs baseline: 1.0764x; 1.0382x over previous
import functools

import jax
import jax.numpy as jnp
from jax import lax
from jax.experimental import pallas as pl
from jax.experimental.pallas import tpu as pltpu
from jax.experimental.pallas import tpu_sc as plsc

N = 10000
E = 320000
D = 128
H = 128
MH = 256

NC = 2
NS = 16
L = 16
NW = NC * NS
NPAD = 10240
RPT = NPAD // NS
EPW = E // NW
K = 80
NCHUNK = EPW // K
ZR = 80
BR = 2048

_mesh = plsc.VectorSubcoreMesh(core_axis_name="c", subcore_axis_name="s",
                               num_cores=NC, num_subcores=NS)


def _seg_body(with_cnt, x_hbm, src_hbm, dst_hbm, pagg_hbm, pcnt_hbm,
              sagg, scnt, src_all, dst_all, rows_v0, rows_v1,
              z1d_v, ones_v, sg0, ss0, sc0, sg1, ss1, sc1):
    cid = lax.axis_index("c")
    sid = lax.axis_index("s")
    wid = sid * NC + cid
    base_r = sid * RPT

    pltpu.sync_copy(src_hbm.at[wid], src_all)
    pltpu.sync_copy(dst_hbm.at[wid], dst_all)

    def zrow_body(r, c):
        for cb in range(D // L):
            rows_v0[r, pl.ds(cb * L, L)] = jnp.zeros((L,), jnp.float32)
        return c
    lax.fori_loop(0, K, zrow_body, 0)
    if with_cnt:
        def z1_body(r, c):
            z1d_v[pl.ds(r * L, L)] = jnp.zeros((L,), jnp.float32)
            return c
        lax.fori_loop(0, RPT // L, z1_body, 0)
        for i in range(K // L):
            ones_v[pl.ds(i * L, L)] = jnp.ones((L,), jnp.float32)

    for i in range(RPT // K):
        pltpu.sync_copy(rows_v0, sagg.at[pl.ds(base_r + i * K, K)])
    if with_cnt:
        pltpu.sync_copy(z1d_v, scnt.at[pl.ds(base_r, RPT)])
    plsc.subcore_barrier()

    bufs = ((rows_v0, sg0, ss0, sc0), (rows_v1, sg1, ss1, sc1))

    def start_gather(j, buf):
        rv, sg, ss, sc = buf
        pltpu.async_copy(x_hbm.at[src_all.at[j]], rv, sg)

    def drain_scatter(j, buf):
        rv, sg, ss, sc = buf
        pltpu.make_async_copy(rv, sagg.at[dst_all.at[j]], ss).wait()
        if with_cnt:
            pltpu.make_async_copy(ones_v, scnt.at[dst_all.at[j]], sc).wait()

    def process(j, buf):
        rv, sg, ss, sc = buf
        pltpu.make_async_copy(x_hbm.at[src_all.at[j]], rv, sg).wait()
        pltpu.async_copy(rv, sagg.at[dst_all.at[j]], ss, add=True)
        if with_cnt:
            pltpu.async_copy(ones_v, scnt.at[dst_all.at[j]], sc, add=True)

    start_gather(0, bufs[0])
    start_gather(1, bufs[1])

    def pair(it, c):
        j0 = 2 * it
        process(j0, bufs[0])
        process(j0 + 1, bufs[1])
        drain_scatter(j0, bufs[0])
        start_gather(j0 + 2, bufs[0])
        drain_scatter(j0 + 1, bufs[1])
        start_gather(j0 + 3, bufs[1])
        return c
    lax.fori_loop(0, (NCHUNK - 3) // 2, pair, 0)
    process(NCHUNK - 3, bufs[0])
    drain_scatter(NCHUNK - 3, bufs[0])
    start_gather(NCHUNK - 1, bufs[0])
    process(NCHUNK - 2, bufs[1])
    process(NCHUNK - 1, bufs[0])
    drain_scatter(NCHUNK - 1, bufs[0])
    drain_scatter(NCHUNK - 2, bufs[1])
    plsc.subcore_barrier()

    out_r = cid * NPAD + base_r
    pltpu.sync_copy(sagg.at[pl.ds(base_r, RPT)], pagg_hbm.at[pl.ds(out_r, RPT)])
    if with_cnt:
        pltpu.sync_copy(scnt.at[pl.ds(base_r, RPT)], pcnt_hbm.at[pl.ds(out_r, RPT)])


def _make_seg(with_cnt):
    outs = (jax.ShapeDtypeStruct((2 * NPAD, D), jnp.float32),
            jax.ShapeDtypeStruct((2 * NPAD,), jnp.float32))
    scratch = [
        pltpu.VMEM_SHARED((NPAD, D), jnp.float32),
        pltpu.VMEM_SHARED((NPAD,), jnp.float32),
        pltpu.VMEM((NCHUNK, K), jnp.int32),
        pltpu.VMEM((NCHUNK, K), jnp.int32),
        pltpu.VMEM((K, D), jnp.float32),
        pltpu.VMEM((K, D), jnp.float32),
        pltpu.VMEM((RPT,), jnp.float32),
        pltpu.VMEM((K,), jnp.float32),
    ] + [pltpu.SemaphoreType.DMA] * 6
    return pl.kernel(functools.partial(_seg_body, with_cnt),
                     out_type=outs, mesh=_mesh, scratch_types=scratch,
                     compiler_params=pltpu.CompilerParams(
                         use_tc_tiling_on_sc=False))


_seg_with_cnt = _make_seg(True)
_seg_no_cnt = _make_seg(False)


def _edge_body(a_hbm, b_hbm, src_hbm, dst_hbm, w2_hbm, bm2_hbm, out_hbm,
               a_v0, b_v0, a_v1, b_v1, a_v2, b_v2, src_all, dst_all,
               out_v0, out_v1, out_v2, w_v, bm2_v, t_v,
               sa0, sb0, sa1, sb1, sa2, sb2, so0, so1, so2):
    cid = lax.axis_index("c")
    sid = lax.axis_index("s")
    wid = sid * NC + cid
    ebase = wid * EPW

    pltpu.sync_copy(src_hbm.at[wid], src_all)
    pltpu.sync_copy(dst_hbm.at[wid], dst_all)
    pltpu.sync_copy(w2_hbm, w_v)
    pltpu.sync_copy(bm2_hbm, bm2_v)
    bias16 = bm2_v[...]
    iota17 = lax.iota(jnp.int32, L) * 17

    bufs = ((a_v0, b_v0, out_v0, sa0, sb0, so0),
            (a_v1, b_v1, out_v1, sa1, sb1, so1),
            (a_v2, b_v2, out_v2, sa2, sb2, so2))

    def start(j, buf):
        av, bv, ov, sa, sb, so = buf
        pltpu.async_copy(a_hbm.at[src_all.at[j]], av, sa)
        pltpu.async_copy(b_hbm.at[dst_all.at[j]], bv, sb)

    def finish(j, buf):
        av, bv, ov, sa, sb, so = buf
        off = ebase + j * K
        pltpu.make_async_copy(a_hbm.at[src_all.at[j]], av, sa).wait()
        pltpu.make_async_copy(b_hbm.at[dst_all.at[j]], bv, sb).wait()
        pltpu.make_async_copy(ov, out_hbm.at[pl.ds(off, K)], so).wait()

        def group(g, c2):
            goff = g * L
            def jb_body(jb, accs):
                col = jb * (2 * L)
                w32 = w_v[pl.ds(col, 2 * L)]
                nxt = []
                for e in range(L):
                    a32 = av[goff + e, pl.ds(col, 2 * L)]
                    b32 = bv[goff + e, pl.ds(col, 2 * L)]
                    p = jnp.maximum(a32 + b32, jnp.bfloat16(0)) * w32
                    p0, p1 = plsc.unpack(p, format=plsc.PackFormat.INTERLEAVED)
                    nxt.append(accs[e] + p0 + p1)
                return tuple(nxt)
            accs = lax.fori_loop(
                0, MH // (2 * L), jb_body,
                tuple(jnp.zeros((L,), jnp.float32) for _ in range(L)))
            for e in range(L):
                t_v[pl.ds(e * 17, L)] = accs[e]
            tot = jnp.zeros((L,), jnp.float32)
            for jj in range(L):
                tot = tot + plsc.load_gather(t_v, [iota17 + jj])
            ov[pl.ds(goff, L)] = tot + bias16
            return c2
        lax.fori_loop(0, K // L, group, 0)
        pltpu.async_copy(ov, out_hbm.at[pl.ds(off, K)], so)

    pltpu.async_copy(out_v0, out_hbm.at[pl.ds(ebase, K)], so0)
    pltpu.async_copy(out_v1, out_hbm.at[pl.ds(ebase + K, K)], so1)
    pltpu.async_copy(out_v2, out_hbm.at[pl.ds(ebase + 2 * K, K)], so2)

    start(0, bufs[0])
    start(1, bufs[1])
    start(2, bufs[2])
    def triple(it, c):
        j0 = 3 * it
        finish(j0, bufs[0])
        start(j0 + 3, bufs[0])
        finish(j0 + 1, bufs[1])
        start(j0 + 4, bufs[1])
        finish(j0 + 2, bufs[2])
        start(j0 + 5, bufs[2])
        return c
    lax.fori_loop(0, 40, triple, 0)
    finish(120, bufs[0])
    start(123, bufs[0])
    finish(121, bufs[1])
    start(124, bufs[1])
    finish(122, bufs[2])
    finish(123, bufs[0])
    finish(124, bufs[1])
    pltpu.make_async_copy(out_v0, out_hbm.at[pl.ds(ebase, K)], so0).wait()
    pltpu.make_async_copy(out_v1, out_hbm.at[pl.ds(ebase, K)], so1).wait()
    pltpu.make_async_copy(out_v2, out_hbm.at[pl.ds(ebase, K)], so2).wait()


_edge_head = pl.kernel(
    _edge_body,
    out_type=jax.ShapeDtypeStruct((E,), jnp.float32),
    mesh=_mesh,
    scratch_types=[
        pltpu.VMEM((K, MH), jnp.bfloat16),
        pltpu.VMEM((K, MH), jnp.bfloat16),
        pltpu.VMEM((K, MH), jnp.bfloat16),
        pltpu.VMEM((K, MH), jnp.bfloat16),
        pltpu.VMEM((K, MH), jnp.bfloat16),
        pltpu.VMEM((K, MH), jnp.bfloat16),
        pltpu.VMEM((NCHUNK, K), jnp.int32),
        pltpu.VMEM((NCHUNK, K), jnp.int32),
        pltpu.VMEM((K,), jnp.float32),
        pltpu.VMEM((K,), jnp.float32),
        pltpu.VMEM((K,), jnp.float32),
        pltpu.VMEM((MH,), jnp.bfloat16),
        pltpu.VMEM((L,), jnp.float32),
        pltpu.VMEM((L * 17,), jnp.float32),
    ] + [pltpu.SemaphoreType.DMA] * 9,
    compiler_params=pltpu.CompilerParams(use_tc_tiling_on_sc=False,
                                         needs_layout_passes=False),
)


def _tc1_body(pagg_ref, pcnt_ref, x_ref, w1lT_ref, b1l_ref, w1rT_ref, h1_ref):
    cnt = pcnt_ref[0] + pcnt_ref[1]
    inv = 1.0 / jnp.maximum(cnt, 1.0)
    agg = (pagg_ref[0] + pagg_ref[1]) * inv
    h = jnp.dot(agg, w1lT_ref[...], preferred_element_type=jnp.float32)
    h = h + jnp.dot(x_ref[...], w1rT_ref[...], preferred_element_type=jnp.float32)
    h1_ref[...] = jnp.maximum(h + b1l_ref[...], 0.0)


def _tc2_body(pagg_ref, pcnt_ref, h1_ref, w2lT_ref, b2l_ref, w2rT_ref,
              wm1aT_ref, wm1bT_ref, bm1_ref, a_ref, b_ref):
    cnt = pcnt_ref[0] + pcnt_ref[1]
    inv = 1.0 / jnp.maximum(cnt, 1.0)
    agg = (pagg_ref[0] + pagg_ref[1]) * inv
    h = jnp.dot(agg, w2lT_ref[...], preferred_element_type=jnp.float32)
    h = h + jnp.dot(h1_ref[...], w2rT_ref[...], preferred_element_type=jnp.float32)
    h2 = jnp.maximum(h + b2l_ref[...], 0.0)
    a_ref[...] = jnp.dot(
        h2, wm1aT_ref[...],
        preferred_element_type=jnp.float32).astype(jnp.bfloat16)
    b_ref[...] = (jnp.dot(h2, wm1bT_ref[...], preferred_element_type=jnp.float32)
                  + bm1_ref[...]).astype(jnp.bfloat16)


def _tc1(pagg, pcnt, x, w1lT, b1l, w1rT):
    grid = NPAD // BR
    return pl.pallas_call(
        _tc1_body,
        grid=(grid,),
        in_specs=[
            pl.BlockSpec((2, BR, D), lambda i: (0, i, 0)),
            pl.BlockSpec((2, BR, 1), lambda i: (0, i, 0)),
            pl.BlockSpec((BR, D), lambda i: (i, 0)),
            pl.BlockSpec((D, H), lambda i: (0, 0)),
            pl.BlockSpec((1, H), lambda i: (0, 0)),
            pl.BlockSpec((D, H), lambda i: (0, 0)),
        ],
        out_specs=pl.BlockSpec((BR, H), lambda i: (i, 0)),
        out_shape=jax.ShapeDtypeStruct((NPAD, H), jnp.float32),
    )(pagg, pcnt, x, w1lT, b1l, w1rT)


def _tc2(pagg, pcnt, h1, w2lT, b2l, w2rT, wm1aT, wm1bT, bm1):
    grid = NPAD // BR
    return pl.pallas_call(
        _tc2_body,
        grid=(grid,),
        in_specs=[
            pl.BlockSpec((2, BR, H), lambda i: (0, i, 0)),
            pl.BlockSpec((2, BR, 1), lambda i: (0, i, 0)),
            pl.BlockSpec((BR, H), lambda i: (i, 0)),
            pl.BlockSpec((H, H), lambda i: (0, 0)),
            pl.BlockSpec((1, H), lambda i: (0, 0)),
            pl.BlockSpec((H, H), lambda i: (0, 0)),
            pl.BlockSpec((H, MH), lambda i: (0, 0)),
            pl.BlockSpec((H, MH), lambda i: (0, 0)),
            pl.BlockSpec((1, MH), lambda i: (0, 0)),
        ],
        out_specs=[
            pl.BlockSpec((BR, MH), lambda i: (i, 0)),
            pl.BlockSpec((BR, MH), lambda i: (i, 0)),
        ],
        out_shape=[
            jax.ShapeDtypeStruct((NPAD, MH), jnp.bfloat16),
            jax.ShapeDtypeStruct((NPAD, MH), jnp.bfloat16),
        ],
    )(pagg, pcnt, h1, w2lT, b2l, w2rT, wm1aT, wm1bT, bm1)


def kernel(x, edge_index, W1l, b1l, W1r, W2l, b2l, W2r, Wm1, bm1, Wm2, bm2):
    src = edge_index[0].reshape(NW, NCHUNK, K)
    dst = edge_index[1].reshape(NW, NCHUNK, K)

    pagg1, pcnt = _seg_with_cnt(x, src, dst)
    pagg1 = pagg1.reshape(2, NPAD, D)
    pcnt2 = pcnt.reshape(2, NPAD, 1)

    xp = jnp.pad(x, ((0, NPAD - N), (0, 0)))
    h1 = _tc1(pagg1, pcnt2, xp, W1l.T, b1l.reshape(1, H), W1r.T)

    pagg2, _ = _seg_no_cnt(h1, src, dst)
    pagg2 = pagg2.reshape(2, NPAD, H)

    A, B = _tc2(pagg2, pcnt2, h1, W2l.T, b2l.reshape(1, H), W2r.T,
                Wm1[:, :H].T, Wm1[:, H:].T, bm1.reshape(1, MH))

    w2 = Wm2[0].astype(jnp.bfloat16)
    bm2s = jnp.full((L,), bm2[0], jnp.float32)
    out = _edge_head(A, B, src, dst, w2, bm2s)
    return out


# speedup vs baseline: 1.1178x; 1.0385x over previous
import functools

import jax
import jax.numpy as jnp
from jax import lax
from jax.experimental import pallas as pl
from jax.experimental.pallas import tpu as pltpu
from jax.experimental.pallas import tpu_sc as plsc

N = 10000
E = 320000
D = 128
H = 128
MH = 256

NC = 2
NS = 16
L = 16
NW = NC * NS
NPAD = 10240
RPT = NPAD // NS
EPW = E // NW
K = 80
NCHUNK = EPW // K
KS = 40
NCHUNKS = EPW // KS
ZR = 80
BR = 2048

_mesh = plsc.VectorSubcoreMesh(core_axis_name="c", subcore_axis_name="s",
                               num_cores=NC, num_subcores=NS)


def _seg_body(with_cnt, x_hbm, src_hbm, dst_hbm, pagg_hbm, pcnt_hbm,
              sagg, scnt, src_all, dst_all, rows_v0, rows_v1, rows_v2,
              z1d_v, ones_v, sg0, ss0, sc0, sg1, ss1, sc1, sg2, ss2, sc2):
    cid = lax.axis_index("c")
    sid = lax.axis_index("s")
    wid = sid * NC + cid
    base_r = sid * RPT

    pltpu.sync_copy(src_hbm.at[wid], src_all)
    pltpu.sync_copy(dst_hbm.at[wid], dst_all)

    def zrow_body(r, c):
        for cb in range(D // L):
            rows_v0[r, pl.ds(cb * L, L)] = jnp.zeros((L,), jnp.float32)
        return c
    lax.fori_loop(0, KS, zrow_body, 0)
    if with_cnt:
        def z1_body(r, c):
            z1d_v[pl.ds(r * L, L)] = jnp.zeros((L,), jnp.float32)
            return c
        lax.fori_loop(0, RPT // L, z1_body, 0)
        for i in range(3):
            ones_v[pl.ds(i * L, L)] = jnp.ones((L,), jnp.float32)

    for i in range(RPT // KS):
        pltpu.sync_copy(rows_v0, sagg.at[pl.ds(base_r + i * KS, KS)])
    if with_cnt:
        pltpu.sync_copy(z1d_v, scnt.at[pl.ds(base_r, RPT)])
    plsc.subcore_barrier()

    ones_s = ones_v.at[pl.ds(0, KS)]
    bufs = ((rows_v0, sg0, ss0, sc0), (rows_v1, sg1, ss1, sc1),
            (rows_v2, sg2, ss2, sc2))

    def start_gather(j, buf):
        rv, sg, ss, sc = buf
        pltpu.async_copy(x_hbm.at[src_all.at[j]], rv, sg)

    def process(j, buf):
        rv, sg, ss, sc = buf
        pltpu.make_async_copy(x_hbm.at[src_all.at[j]], rv, sg).wait()
        pltpu.async_copy(rv, sagg.at[dst_all.at[j]], ss, add=True)
        if with_cnt:
            pltpu.async_copy(ones_s, scnt.at[dst_all.at[j]], sc, add=True)

    def drain(j, buf):
        rv, sg, ss, sc = buf
        pltpu.make_async_copy(rv, sagg.at[dst_all.at[j]], ss).wait()
        if with_cnt:
            pltpu.make_async_copy(ones_s, scnt.at[dst_all.at[j]], sc).wait()

    start_gather(0, bufs[0])
    start_gather(1, bufs[1])
    start_gather(2, bufs[2])

    def triple(it, c):
        j0 = 3 * it
        process(j0, bufs[0])
        process(j0 + 1, bufs[1])
        process(j0 + 2, bufs[2])
        drain(j0, bufs[0])
        start_gather(j0 + 3, bufs[0])
        drain(j0 + 1, bufs[1])
        start_gather(j0 + 4, bufs[1])
        drain(j0 + 2, bufs[2])
        start_gather(j0 + 5, bufs[2])
        return c
    lax.fori_loop(0, 82, triple, 0)
    process(246, bufs[0])
    process(247, bufs[1])
    process(248, bufs[2])
    drain(246, bufs[0])
    start_gather(249, bufs[0])
    drain(247, bufs[1])
    drain(248, bufs[2])
    process(249, bufs[0])
    drain(249, bufs[0])
    plsc.subcore_barrier()

    out_r = cid * NPAD + base_r
    pltpu.sync_copy(sagg.at[pl.ds(base_r, RPT)], pagg_hbm.at[pl.ds(out_r, RPT)])
    if with_cnt:
        pltpu.sync_copy(scnt.at[pl.ds(base_r, RPT)], pcnt_hbm.at[pl.ds(out_r, RPT)])


def _make_seg(with_cnt):
    outs = (jax.ShapeDtypeStruct((2 * NPAD, D), jnp.float32),
            jax.ShapeDtypeStruct((2 * NPAD,), jnp.float32))
    scratch = [
        pltpu.VMEM_SHARED((NPAD, D), jnp.float32),
        pltpu.VMEM_SHARED((NPAD,), jnp.float32),
        pltpu.VMEM((NCHUNKS, KS), jnp.int32),
        pltpu.VMEM((NCHUNKS, KS), jnp.int32),
        pltpu.VMEM((KS, D), jnp.float32),
        pltpu.VMEM((KS, D), jnp.float32),
        pltpu.VMEM((KS, D), jnp.float32),
        pltpu.VMEM((RPT,), jnp.float32),
        pltpu.VMEM((48,), jnp.float32),
    ] + [pltpu.SemaphoreType.DMA] * 9
    return pl.kernel(functools.partial(_seg_body, with_cnt),
                     out_type=outs, mesh=_mesh, scratch_types=scratch,
                     compiler_params=pltpu.CompilerParams(
                         use_tc_tiling_on_sc=False))


_seg_with_cnt = _make_seg(True)
_seg_no_cnt = _make_seg(False)


def _edge_body(a_hbm, b_hbm, src_hbm, dst_hbm, w2_hbm, bm2_hbm, out_hbm,
               a_v0, b_v0, a_v1, b_v1, a_v2, b_v2, src_all, dst_all,
               out_v0, out_v1, out_v2, w_v, bm2_v, t_v,
               sa0, sb0, sa1, sb1, sa2, sb2, so0, so1, so2):
    cid = lax.axis_index("c")
    sid = lax.axis_index("s")
    wid = sid * NC + cid
    ebase = wid * EPW

    pltpu.sync_copy(src_hbm.at[wid], src_all)
    pltpu.sync_copy(dst_hbm.at[wid], dst_all)
    pltpu.sync_copy(w2_hbm, w_v)
    pltpu.sync_copy(bm2_hbm, bm2_v)
    bias16 = bm2_v[...]
    iota17 = lax.iota(jnp.int32, L) * 17

    bufs = ((a_v0, b_v0, out_v0, sa0, sb0, so0),
            (a_v1, b_v1, out_v1, sa1, sb1, so1),
            (a_v2, b_v2, out_v2, sa2, sb2, so2))

    def start(j, buf):
        av, bv, ov, sa, sb, so = buf
        pltpu.async_copy(a_hbm.at[src_all.at[j]], av, sa)
        pltpu.async_copy(b_hbm.at[dst_all.at[j]], bv, sb)

    def finish(j, buf):
        av, bv, ov, sa, sb, so = buf
        off = ebase + j * K
        pltpu.make_async_copy(a_hbm.at[src_all.at[j]], av, sa).wait()
        pltpu.make_async_copy(b_hbm.at[dst_all.at[j]], bv, sb).wait()
        pltpu.make_async_copy(ov, out_hbm.at[pl.ds(off, K)], so).wait()

        def group(g, c2):
            goff = g * L
            def jb_body(jb, accs):
                col = jb * (2 * L)
                w32 = w_v[pl.ds(col, 2 * L)]
                nxt = []
                for e in range(L):
                    a32 = av[goff + e, pl.ds(col, 2 * L)]
                    b32 = bv[goff + e, pl.ds(col, 2 * L)]
                    p = jnp.maximum(a32 + b32, jnp.bfloat16(0)) * w32
                    p0, p1 = plsc.unpack(p, format=plsc.PackFormat.INTERLEAVED)
                    nxt.append(accs[e] + p0 + p1)
                return tuple(nxt)
            accs = lax.fori_loop(
                0, MH // (2 * L), jb_body,
                tuple(jnp.zeros((L,), jnp.float32) for _ in range(L)))
            for e in range(L):
                t_v[pl.ds(e * 17, L)] = accs[e]
            tot = jnp.zeros((L,), jnp.float32)
            for jj in range(L):
                tot = tot + plsc.load_gather(t_v, [iota17 + jj])
            ov[pl.ds(goff, L)] = tot + bias16
            return c2
        lax.fori_loop(0, K // L, group, 0)
        pltpu.async_copy(ov, out_hbm.at[pl.ds(off, K)], so)

    pltpu.async_copy(out_v0, out_hbm.at[pl.ds(ebase, K)], so0)
    pltpu.async_copy(out_v1, out_hbm.at[pl.ds(ebase + K, K)], so1)
    pltpu.async_copy(out_v2, out_hbm.at[pl.ds(ebase + 2 * K, K)], so2)

    start(0, bufs[0])
    start(1, bufs[1])
    start(2, bufs[2])
    def triple(it, c):
        j0 = 3 * it
        finish(j0, bufs[0])
        start(j0 + 3, bufs[0])
        finish(j0 + 1, bufs[1])
        start(j0 + 4, bufs[1])
        finish(j0 + 2, bufs[2])
        start(j0 + 5, bufs[2])
        return c
    lax.fori_loop(0, 40, triple, 0)
    finish(120, bufs[0])
    start(123, bufs[0])
    finish(121, bufs[1])
    start(124, bufs[1])
    finish(122, bufs[2])
    finish(123, bufs[0])
    finish(124, bufs[1])
    pltpu.make_async_copy(out_v0, out_hbm.at[pl.ds(ebase, K)], so0).wait()
    pltpu.make_async_copy(out_v1, out_hbm.at[pl.ds(ebase, K)], so1).wait()
    pltpu.make_async_copy(out_v2, out_hbm.at[pl.ds(ebase, K)], so2).wait()


_edge_head = pl.kernel(
    _edge_body,
    out_type=jax.ShapeDtypeStruct((E,), jnp.float32),
    mesh=_mesh,
    scratch_types=[
        pltpu.VMEM((K, MH), jnp.bfloat16),
        pltpu.VMEM((K, MH), jnp.bfloat16),
        pltpu.VMEM((K, MH), jnp.bfloat16),
        pltpu.VMEM((K, MH), jnp.bfloat16),
        pltpu.VMEM((K, MH), jnp.bfloat16),
        pltpu.VMEM((K, MH), jnp.bfloat16),
        pltpu.VMEM((NCHUNK, K), jnp.int32),
        pltpu.VMEM((NCHUNK, K), jnp.int32),
        pltpu.VMEM((K,), jnp.float32),
        pltpu.VMEM((K,), jnp.float32),
        pltpu.VMEM((K,), jnp.float32),
        pltpu.VMEM((MH,), jnp.bfloat16),
        pltpu.VMEM((L,), jnp.float32),
        pltpu.VMEM((L * 17,), jnp.float32),
    ] + [pltpu.SemaphoreType.DMA] * 9,
    compiler_params=pltpu.CompilerParams(use_tc_tiling_on_sc=False,
                                         needs_layout_passes=False),
)


def _tc1_body(pagg_ref, pcnt_ref, x_ref, w1lT_ref, b1l_ref, w1rT_ref, h1_ref):
    cnt = pcnt_ref[0] + pcnt_ref[1]
    inv = 1.0 / jnp.maximum(cnt, 1.0)
    agg = (pagg_ref[0] + pagg_ref[1]) * inv
    h = jnp.dot(agg, w1lT_ref[...], preferred_element_type=jnp.float32)
    h = h + jnp.dot(x_ref[...], w1rT_ref[...], preferred_element_type=jnp.float32)
    h1_ref[...] = jnp.maximum(h + b1l_ref[...], 0.0)


def _tc2_body(pagg_ref, pcnt_ref, h1_ref, w2lT_ref, b2l_ref, w2rT_ref,
              wm1aT_ref, wm1bT_ref, bm1_ref, a_ref, b_ref):
    cnt = pcnt_ref[0] + pcnt_ref[1]
    inv = 1.0 / jnp.maximum(cnt, 1.0)
    agg = (pagg_ref[0] + pagg_ref[1]) * inv
    h = jnp.dot(agg, w2lT_ref[...], preferred_element_type=jnp.float32)
    h = h + jnp.dot(h1_ref[...], w2rT_ref[...], preferred_element_type=jnp.float32)
    h2 = jnp.maximum(h + b2l_ref[...], 0.0)
    a_ref[...] = jnp.dot(
        h2, wm1aT_ref[...],
        preferred_element_type=jnp.float32).astype(jnp.bfloat16)
    b_ref[...] = (jnp.dot(h2, wm1bT_ref[...], preferred_element_type=jnp.float32)
                  + bm1_ref[...]).astype(jnp.bfloat16)


def _tc1(pagg, pcnt, x, w1lT, b1l, w1rT):
    grid = NPAD // BR
    return pl.pallas_call(
        _tc1_body,
        grid=(grid,),
        in_specs=[
            pl.BlockSpec((2, BR, D), lambda i: (0, i, 0)),
            pl.BlockSpec((2, BR, 1), lambda i: (0, i, 0)),
            pl.BlockSpec((BR, D), lambda i: (i, 0)),
            pl.BlockSpec((D, H), lambda i: (0, 0)),
            pl.BlockSpec((1, H), lambda i: (0, 0)),
            pl.BlockSpec((D, H), lambda i: (0, 0)),
        ],
        out_specs=pl.BlockSpec((BR, H), lambda i: (i, 0)),
        out_shape=jax.ShapeDtypeStruct((NPAD, H), jnp.float32),
    )(pagg, pcnt, x, w1lT, b1l, w1rT)


def _tc2(pagg, pcnt, h1, w2lT, b2l, w2rT, wm1aT, wm1bT, bm1):
    grid = NPAD // BR
    return pl.pallas_call(
        _tc2_body,
        grid=(grid,),
        in_specs=[
            pl.BlockSpec((2, BR, H), lambda i: (0, i, 0)),
            pl.BlockSpec((2, BR, 1), lambda i: (0, i, 0)),
            pl.BlockSpec((BR, H), lambda i: (i, 0)),
            pl.BlockSpec((H, H), lambda i: (0, 0)),
            pl.BlockSpec((1, H), lambda i: (0, 0)),
            pl.BlockSpec((H, H), lambda i: (0, 0)),
            pl.BlockSpec((H, MH), lambda i: (0, 0)),
            pl.BlockSpec((H, MH), lambda i: (0, 0)),
            pl.BlockSpec((1, MH), lambda i: (0, 0)),
        ],
        out_specs=[
            pl.BlockSpec((BR, MH), lambda i: (i, 0)),
            pl.BlockSpec((BR, MH), lambda i: (i, 0)),
        ],
        out_shape=[
            jax.ShapeDtypeStruct((NPAD, MH), jnp.bfloat16),
            jax.ShapeDtypeStruct((NPAD, MH), jnp.bfloat16),
        ],
    )(pagg, pcnt, h1, w2lT, b2l, w2rT, wm1aT, wm1bT, bm1)


def kernel(x, edge_index, W1l, b1l, W1r, W2l, b2l, W2r, Wm1, bm1, Wm2, bm2):
    src = edge_index[0].reshape(NW, NCHUNK, K)
    dst = edge_index[1].reshape(NW, NCHUNK, K)
    src_s = edge_index[0].reshape(NW, NCHUNKS, KS)
    dst_s = edge_index[1].reshape(NW, NCHUNKS, KS)

    pagg1, pcnt = _seg_with_cnt(x, src_s, dst_s)
    pagg1 = pagg1.reshape(2, NPAD, D)
    pcnt2 = pcnt.reshape(2, NPAD, 1)

    xp = jnp.pad(x, ((0, NPAD - N), (0, 0)))
    h1 = _tc1(pagg1, pcnt2, xp, W1l.T, b1l.reshape(1, H), W1r.T)

    pagg2, _ = _seg_no_cnt(h1, src_s, dst_s)
    pagg2 = pagg2.reshape(2, NPAD, H)

    A, B = _tc2(pagg2, pcnt2, h1, W2l.T, b2l.reshape(1, H), W2r.T,
                Wm1[:, :H].T, Wm1[:, H:].T, bm1.reshape(1, MH))

    w2 = Wm2[0].astype(jnp.bfloat16)
    bm2s = jnp.full((L,), bm2[0], jnp.float32)
    out = _edge_head(A, B, src, dst, w2, bm2s)
    return out
